# baseline (speedup 1.0000x reference)
import jax
import jax.numpy as jnp
from jax import lax
from jax.experimental import pallas as pl
from jax.experimental.pallas import tpu as pltpu
from jax.experimental.pallas import tpu_sc as plsc

_MAX_LEN = 200
_DIM = 64
_BATCH = 4096

_NC, _NS = 2, 16
_NW = _NC * _NS
_SPW = _BATCH // _NW
_CHUNK = _MAX_LEN // 2
_NCHUNK = 2 * _SPW
_NBUF = 4


def _pe_table():
    pos = jnp.arange(_MAX_LEN, dtype=jnp.float32).reshape(-1, 1)
    dim = jnp.arange(_DIM, dtype=jnp.float32).reshape(1, -1)
    phase = pos / (10000.0 ** (dim / _DIM))
    return jnp.where((jnp.arange(_DIM) % 2) == 0, jnp.sin(phase), jnp.cos(phase))


def _body(tok_hbm, pe_hbm, table_hbm, out_hbm, idx_v, pe_sh, *rest):
    sid = lax.axis_index("s")
    wid = sid * _NC + lax.axis_index("c")
    chunk0 = wid * _NCHUNK
    seq0 = wid * _SPW

    pltpu.sync_copy(tok_hbm.at[pl.ds(chunk0, _NCHUNK)], idx_v)

    @pl.when(sid == 0)
    def _():
        pltpu.sync_copy(pe_hbm, pe_sh)

    plsc.subcore_barrier()

    bufs = rest[:_NBUF]
    gsems = rest[_NBUF:2 * _NBUF]
    ssems = rest[2 * _NBUF:3 * _NBUF]

    def _pe_fill(c, b):
        pos0 = (b % 2) * _CHUNK
        pltpu.sync_copy(pe_sh.at[pl.ds(pos0, _CHUNK)], bufs[b])

    def _gather_start(c, b):
        pltpu.async_copy(table_hbm.at[idx_v.at[c]], bufs[b], gsems[b], add=True)

    def _gather_wait(c, b):
        pltpu.make_async_copy(table_hbm.at[idx_v.at[c]], bufs[b], gsems[b]).wait()

    def _out_slice(c, b):
        return out_hbm.at[seq0 + lax.div(c, 2), pl.ds((b % 2) * _CHUNK, _CHUNK)]

    def _scatter_start(c, b):
        pltpu.async_copy(bufs[b], _out_slice(c, b), ssems[b])

    def _scatter_wait(c, b):
        pltpu.make_async_copy(bufs[b], _out_slice(c, b), ssems[b]).wait()

    for c in range(_NBUF - 1):
        _pe_fill(c, c)
        _gather_start(c, c)

    def outer(g, _):
        for b in range(_NBUF):
            c = g * _NBUF + b
            _gather_wait(c, b)
            _scatter_start(c, b)

            cn = c + (_NBUF - 1)
            bn = (b + _NBUF - 1) % _NBUF

            @pl.when(cn < _NCHUNK)
            def _():
                @pl.when(c >= 1)
                def _():
                    _scatter_wait(c - 1, bn)

                _pe_fill(cn, bn)
                _gather_start(cn, bn)

        return 0

    lax.fori_loop(0, _NCHUNK // _NBUF, outer, 0)

    for b in range(_NBUF):
        _scatter_wait(_NCHUNK - _NBUF + b, b)


_mesh = plsc.VectorSubcoreMesh(core_axis_name="c", subcore_axis_name="s")

_sc_call = pl.kernel(
    _body,
    out_type=jax.ShapeDtypeStruct((_BATCH, _MAX_LEN, _DIM), jnp.float32),
    mesh=_mesh,
    scratch_types=[
        pltpu.VMEM((_NCHUNK, _CHUNK), jnp.int32),
        pltpu.VMEM_SHARED((_MAX_LEN, _DIM), jnp.float32),
        *([pltpu.VMEM((_CHUNK, _DIM), jnp.float32)] * _NBUF),
        *([pltpu.SemaphoreType.DMA] * _NBUF),
        *([pltpu.SemaphoreType.DMA] * _NBUF),
    ],
    compiler_params=pltpu.CompilerParams(use_tc_tiling_on_sc=False),
)


def kernel(tokens, table):
    tok = tokens.astype(jnp.int32).reshape(_BATCH * 2, _CHUNK)
    pe = _pe_table()
    return _sc_call(tok, pe, table)


# speedup vs baseline: 1.3297x; 1.3297x over previous
import jax
import jax.numpy as jnp
from jax import lax
from jax.experimental import pallas as pl
from jax.experimental.pallas import tpu as pltpu
from jax.experimental.pallas import tpu_sc as plsc

_MAX_LEN = 200
_DIM = 64
_BATCH = 4096

_NC, _NS, _L = 2, 16, 16
_NW = _NC * _NS
_ROWS = _BATCH * _MAX_LEN
_RPW = _ROWS // _NW
_CHUNK = 128
_NCHUNK = _RPW // _CHUNK


def _pe_doubled():
    pos = jnp.arange(_MAX_LEN, dtype=jnp.float32).reshape(-1, 1)
    dim = jnp.arange(_DIM, dtype=jnp.float32).reshape(1, -1)
    phase = pos / (10000.0 ** (dim / _DIM))
    pe = jnp.where((jnp.arange(_DIM) % 2) == 0, jnp.sin(phase), jnp.cos(phase))
    pe = jnp.concatenate([pe, pe], axis=0)
    return jnp.pad(pe, ((0, 0), (0, _DIM)))


_NBUF = 4


def _body(tok_hbm, pe_hbm, table_hbm, out_hbm, idx_v, pe_sh, *rest):
    sid = lax.axis_index("s")
    wid = sid * _NC + lax.axis_index("c")
    chunk0 = wid * _NCHUNK
    row0 = wid * _RPW

    pltpu.sync_copy(tok_hbm.at[pl.ds(chunk0, _NCHUNK)], idx_v)

    @pl.when(sid == 0)
    def _():
        pltpu.sync_copy(pe_hbm, pe_sh)

    plsc.subcore_barrier()

    bufs = rest[:_NBUF]
    gsems = rest[_NBUF:2 * _NBUF]
    ssems = rest[2 * _NBUF:3 * _NBUF]

    def _pe_fill(c, b):
        pos0 = lax.rem(c * _CHUNK, _MAX_LEN)
        pltpu.sync_copy(pe_sh.at[pl.ds(pos0, _CHUNK)], bufs[b])

    def _gather_start(c, b):
        pltpu.async_copy(table_hbm.at[idx_v.at[c]], bufs[b], gsems[b], add=True)

    def _gather_wait(c, b):
        pltpu.make_async_copy(table_hbm.at[idx_v.at[c]], bufs[b], gsems[b]).wait()

    def _scatter_start(c, b):
        pltpu.async_copy(
            bufs[b], out_hbm.at[pl.ds(row0 + c * _CHUNK, _CHUNK)], ssems[b]
        )

    def _scatter_wait(c, b):
        pltpu.make_async_copy(
            bufs[b], out_hbm.at[pl.ds(row0 + c * _CHUNK, _CHUNK)], ssems[b]
        ).wait()

    for c in range(_NBUF - 1):
        _pe_fill(c, c)
        _gather_start(c, c)

    def outer(g, _):
        for b in range(_NBUF):
            c = g * _NBUF + b
            _gather_wait(c, b)
            _scatter_start(c, b)

            cn = c + (_NBUF - 1)
            bn = (b + _NBUF - 1) % _NBUF

            @pl.when(cn < _NCHUNK)
            def _():
                @pl.when(c >= 1)
                def _():
                    _scatter_wait(c - 1, bn)

                _pe_fill(cn, bn)
                _gather_start(cn, bn)

        return 0

    lax.fori_loop(0, _NCHUNK // _NBUF, outer, 0)

    for b in range(_NBUF):
        _scatter_wait(_NCHUNK - _NBUF + b, b)


_mesh = plsc.VectorSubcoreMesh(core_axis_name="c", subcore_axis_name="s")

_sc_call = pl.kernel(
    _body,
    out_type=jax.ShapeDtypeStruct((_ROWS, 2 * _DIM), jnp.float32),
    mesh=_mesh,
    scratch_types=[
        pltpu.VMEM((_NCHUNK, _CHUNK), jnp.int32),
        pltpu.VMEM_SHARED((2 * _MAX_LEN, 2 * _DIM), jnp.float32),
        *([pltpu.VMEM((_CHUNK, 2 * _DIM), jnp.float32)] * _NBUF),
        *([pltpu.SemaphoreType.DMA] * _NBUF),
        *([pltpu.SemaphoreType.DMA] * _NBUF),
    ],
    compiler_params=pltpu.CompilerParams(use_tc_tiling_on_sc=True),
)


def kernel(tokens, table):
    tok = tokens.astype(jnp.int32).reshape(_ROWS // _CHUNK, _CHUNK)
    pe = _pe_doubled()
    table_p = jnp.pad(table, ((0, 0), (0, _DIM)))
    out = _sc_call(tok, pe, table_p)
    return out[:, :_DIM].reshape(_BATCH, _MAX_LEN, _DIM)
